# baseline (speedup 1.0000x reference)
import functools

import jax
import jax.numpy as jnp
from jax import lax
from jax.experimental import pallas as pl
from jax.experimental.pallas import tpu as pltpu
from jax.experimental.pallas import tpu_sc as plsc

B, A, H, W = 32, 9, 37, 62
RP = B * A * H * 2
RR = B * A * H * 4
JOB = 296
STG = 148
NJP = RP // JOB
COLS = (0, 16, 32, 46)

_mesh = plsc.VectorSubcoreMesh(core_axis_name="c", subcore_axis_name="s")


@functools.partial(
    pl.kernel,
    out_type=jax.ShapeDtypeStruct((RP, W), jnp.float32),
    mesh=_mesh,
    scratch_types=[
        pltpu.VMEM((4, H, W), jnp.float32),
        pltpu.VMEM((4, H, W), jnp.float32),
        pltpu.VMEM((2 * JOB, W), jnp.float32),
        pltpu.SemaphoreType.DMA,
        pltpu.SemaphoreType.DMA,
        pltpu.SemaphoreType.DMA,
    ],
)
def _sc_preds(preds_hbm, outp_hbm, pbuf0, pbuf1, obuf, sg0, sg1, so):
    w = lax.axis_index("s") * 2 + lax.axis_index("c")
    pbufs, sgs = (pbuf0, pbuf1), (sg0, sg1)
    extra = w < RP // (2 * JOB) - 32

    def make_gathers(i, base_u):
        pbuf, sg = pbufs[i % 2], sgs[i % 2]
        cps = []
        for k in range(4):
            u = base_u + k // 2
            b = u // A
            a = u - b * A
            cps.append(pltpu.make_async_copy(
                preds_hbm.at[b, (k % 2) * A + a], pbuf.at[k], sg))
        return cps

    def interleave(i, obase):
        pbuf = pbufs[i % 2]

        def body(h, _):
            for k in range(4):
                row = obase + (k // 2) * 74 + h * 2 + (k % 2)
                for col in COLS:
                    obuf[row, pl.ds(col, 16)] = pbuf[k, h, pl.ds(col, 16)]
            return 0
        lax.fori_loop(0, H, body, 0)

    u0 = 8 * w
    g = make_gathers(0, u0)
    for c in g:
        c.start()
    for i in range(4):
        if i + 1 < 4:
            gn = make_gathers(i + 1, u0 + 2 * (i + 1))
            for c in gn:
                c.start()
        else:
            gn = None
        for c in g:
            c.wait()
        g = gn
        interleave(i, i * STG)
    big_out = pltpu.make_async_copy(
        obuf, outp_hbm.at[pl.ds(w * 2 * JOB, 2 * JOB)], so)
    big_out.start()

    ue = 8 * (32 + w)

    @pl.when(extra)
    def _():
        for c in make_gathers(4, ue):
            c.start()
    big_out.wait()

    @pl.when(extra)
    def _():
        g2 = make_gathers(4, ue)
        for i in range(4):
            if i + 1 < 4:
                gn2 = make_gathers(4 + i + 1, ue + 2 * (i + 1))
                for c in gn2:
                    c.start()
            else:
                gn2 = None
            for c in g2:
                c.wait()
            g2 = gn2
            interleave(4 + i, i * STG)
        cp = pltpu.make_async_copy(
            obuf, outp_hbm.at[pl.ds((32 + w) * 2 * JOB, 2 * JOB)], so)
        cp.start()
        cp.wait()


OCT = B // 8


def _tc_relayout_body(x_ref, o_ref):
    o_ref[...] = x_ref[...].transpose(2, 0, 1, 3)


def _tc_relayout(pt4):
    return pl.pallas_call(
        _tc_relayout_body,
        out_shape=jax.ShapeDtypeStruct((B, 2 * A, H, W), jnp.float32),
        grid=(OCT,),
        in_specs=[pl.BlockSpec((2 * A, H, 8, W), lambda q: (0, 0, q, 0))],
        out_specs=pl.BlockSpec((8, 2 * A, H, W), lambda q: (q, 0, 0, 0)),
    )(pt4)


def _tc_regs_body(x_ref, o_ref):
    x = x_ref[...]
    y = x.reshape(4, A, H, 8, W).transpose(3, 1, 2, 0, 4)
    o_ref[...] = y.reshape(RR // OCT, W)


def _tc_regs(rt4):
    return pl.pallas_call(
        _tc_regs_body,
        out_shape=jax.ShapeDtypeStruct((RR, W), jnp.float32),
        grid=(OCT,),
        in_specs=[pl.BlockSpec((4 * A, H, 8, W), lambda q: (0, 0, q, 0))],
        out_specs=pl.BlockSpec((RR // OCT, W), lambda q: (q, 0)),
    )(rt4)


def kernel(preds, regs):
    bs, _, fh, fw = preds.shape
    pt4 = preds.transpose(1, 2, 0, 3)
    rt4 = regs.transpose(1, 2, 0, 3)
    preds_bm = _tc_relayout(pt4)
    outp = _sc_preds(preds_bm)
    outr = _tc_regs(rt4)
    return (
        outp.reshape(bs, A, fh, 2, fw).transpose(0, 1, 2, 4, 3),
        outr.reshape(bs, A, fh, 4, fw).transpose(0, 1, 2, 4, 3),
    )


# speedup vs baseline: 1.1077x; 1.1077x over previous
import functools

import jax
import jax.numpy as jnp
from jax import lax
from jax.experimental import pallas as pl
from jax.experimental.pallas import tpu as pltpu
from jax.experimental.pallas import tpu_sc as plsc

B, A, H, W = 32, 9, 37, 62
RP = B * A * H * 2
RR = B * A * H * 4
JOB = 296
STG = 148
NJP = RP // JOB
COLS = (0, 16, 32, 46)

_mesh = plsc.VectorSubcoreMesh(core_axis_name="c", subcore_axis_name="s")


@functools.partial(
    pl.kernel,
    out_type=jax.ShapeDtypeStruct((RP, W), jnp.float32),
    mesh=_mesh,
    scratch_types=[
        pltpu.VMEM((4, H, W), jnp.float32),
        pltpu.VMEM((4, H, W), jnp.float32),
        pltpu.VMEM((2 * JOB, W), jnp.float32),
        pltpu.SemaphoreType.DMA,
        pltpu.SemaphoreType.DMA,
        pltpu.SemaphoreType.DMA,
    ],
)
def _sc_preds(preds_hbm, outp_hbm, pbuf0, pbuf1, obuf, sg0, sg1, so):
    w = lax.axis_index("s") * 2 + lax.axis_index("c")
    pbufs, sgs = (pbuf0, pbuf1), (sg0, sg1)
    extra = w < NJP - 2 * 32

    def make_gathers(i, base_u):
        pbuf, sg = pbufs[i % 2], sgs[i % 2]
        cps = []
        for k in range(4):
            u = base_u + k // 2
            b = u // A
            a = u - b * A
            cps.append(pltpu.make_async_copy(
                preds_hbm.at[b, (k % 2) * A + a], pbuf.at[k], sg))
        return cps

    def interleave(i, obase):
        pbuf = pbufs[i % 2]

        def body(h, _):
            for k in range(4):
                row = obase + (k // 2) * 74 + h * 2 + (k % 2)
                for col in COLS:
                    obuf[row, pl.ds(col, 16)] = pbuf[k, h, pl.ds(col, 16)]
            return 0
        lax.fori_loop(0, H, body, 0)

    u0 = 8 * w
    g = make_gathers(0, u0 + 0)
    for c in g:
        c.start()
    for i in range(4):
        if i + 1 < 4:
            gn = make_gathers(i + 1, u0 + 2 * (i + 1))
            for c in gn:
                c.start()
        else:
            gn = None
        for c in g:
            c.wait()
        g = gn
        interleave(i, i * STG)
    big_out = pltpu.make_async_copy(
        obuf, outp_hbm.at[pl.ds(w * 2 * JOB, 2 * JOB)], so)
    big_out.start()

    ue = 4 * (64 + w)

    @pl.when(extra)
    def _():
        ge = make_gathers(0, ue)
        for c in ge:
            c.start()
    @pl.when(extra)
    def _():
        ge2 = make_gathers(1, ue + 2)
        for c in ge2:
            c.start()
    big_out.wait()

    @pl.when(extra)
    def _():
        g0 = make_gathers(0, ue)
        for c in g0:
            c.wait()
        interleave(0, 0)
        g1 = make_gathers(1, ue + 2)
        for c in g1:
            c.wait()
        interleave(1, STG)
        cp = pltpu.make_async_copy(
            obuf.at[pl.ds(0, JOB)],
            outp_hbm.at[pl.ds((64 + w) * JOB, JOB)], so)
        cp.start()
        cp.wait()


OCT = B // 8


def _tc_relayout_body(x_ref, o_ref):
    o_ref[...] = x_ref[...].transpose(2, 0, 1, 3)


def _tc_relayout(pt4):
    return pl.pallas_call(
        _tc_relayout_body,
        out_shape=jax.ShapeDtypeStruct((B, 2 * A, H, W), jnp.float32),
        grid=(OCT,),
        in_specs=[pl.BlockSpec((2 * A, H, 8, W), lambda q: (0, 0, q, 0))],
        out_specs=pl.BlockSpec((8, 2 * A, H, W), lambda q: (q, 0, 0, 0)),
    )(pt4)


def _tc_regs_body(x_ref, o_ref):
    x = x_ref[...]
    y = x.reshape(4, A, H, 8, W).transpose(3, 1, 2, 0, 4)
    o_ref[...] = y.reshape(RR // OCT, W)


def _tc_regs(rt4):
    return pl.pallas_call(
        _tc_regs_body,
        out_shape=jax.ShapeDtypeStruct((RR, W), jnp.float32),
        grid=(OCT,),
        in_specs=[pl.BlockSpec((4 * A, H, 8, W), lambda q: (0, 0, q, 0))],
        out_specs=pl.BlockSpec((RR // OCT, W), lambda q: (q, 0)),
    )(rt4)


def kernel(preds, regs):
    bs, _, fh, fw = preds.shape
    pt4 = preds.transpose(1, 2, 0, 3)
    rt4 = regs.transpose(1, 2, 0, 3)
    preds_bm = _tc_relayout(pt4)
    outp = _sc_preds(preds_bm)
    outr = _tc_regs(rt4)
    return (
        outp.reshape(bs, A, fh, 2, fw).transpose(0, 1, 2, 4, 3),
        outr.reshape(bs, A, fh, 4, fw).transpose(0, 1, 2, 4, 3),
    )
